# baseline (speedup 1.0000x reference)
import jax
import jax.numpy as jnp
from jax import lax
from jax.experimental import pallas as pl
from jax.experimental.pallas import tpu as pltpu
from jax.experimental.pallas import tpu_sc as plsc

B = 4
S = 4096
D = 128
NC = 2
NS = 16
NW = NC * NS
SW = S // NW
LANES = 16


def _emb_kernel(ids_hbm, tok_hbm, pos_hbm, out_hbm, idx_v, rows_v, pos_v,
                isem, psem, gsem, ssem):
    wid = lax.axis_index("s") * NC + lax.axis_index("c")
    base = wid * SW

    icp = pltpu.async_copy(ids_hbm.at[:, pl.ds(base, SW)], idx_v, isem)
    pcp = pltpu.async_copy(pos_hbm.at[pl.ds(base, SW)], pos_v, psem)
    pcp.wait()

    gcps = []
    for b in range(B):
        def copy_body(j, carry, b=b):
            for d in range(D // LANES):
                sl = pl.ds(d * LANES, LANES)
                rows_v[b, j, sl] = pos_v[j, sl]
            return carry

        lax.fori_loop(0, SW, copy_body, 0)
        if b == 0:
            icp.wait()
        gcps.append(pltpu.async_copy(tok_hbm.at[idx_v.at[b]],
                                     rows_v.at[b], gsem.at[b], add=True))

    scps = []
    for b in range(B):
        gcps[b].wait()
        scps.append(pltpu.async_copy(rows_v.at[b],
                                     out_hbm.at[b, pl.ds(base, SW), :],
                                     ssem.at[b]))
    for b in range(B):
        scps[b].wait()


@jax.jit
def _emb(input_ids, token_table, pos_table):
    mesh = plsc.VectorSubcoreMesh(core_axis_name="c", subcore_axis_name="s")
    return pl.kernel(
        _emb_kernel,
        mesh=mesh,
        out_type=jax.ShapeDtypeStruct((B, S, D), jnp.float32),
        scratch_types=[
            pltpu.VMEM((B, SW), jnp.int32),
            pltpu.VMEM((B, SW, D), jnp.float32),
            pltpu.VMEM((SW, D), jnp.float32),
            pltpu.SemaphoreType.DMA,
            pltpu.SemaphoreType.DMA,
            pltpu.SemaphoreType.DMA((B,)),
            pltpu.SemaphoreType.DMA((B,)),
        ],
    )(input_ids, token_table, pos_table)


def kernel(input_ids, token_table, pos_table):
    return _emb(input_ids, token_table, pos_table)


# speedup vs baseline: 1.0132x; 1.0132x over previous
import jax
import jax.numpy as jnp
from jax import lax
from jax.experimental import pallas as pl
from jax.experimental.pallas import tpu as pltpu
from jax.experimental.pallas import tpu_sc as plsc

B = 4
S = 4096
D = 128
NC = 2
NS = 16
NW = NC * NS
SW = S // NW
LANES = 16


def _emb_kernel(ids_hbm, tok_hbm, pos_hbm, out_hbm, idx_v, rows_v, pos_v,
                isem, fsem, psem, gsem, ssem):
    wid = lax.axis_index("s") * NC + lax.axis_index("c")
    base = wid * SW

    icp = pltpu.async_copy(ids_hbm.at[:, pl.ds(base, SW)], idx_v, isem)
    f0p = pltpu.async_copy(pos_hbm.at[pl.ds(base, SW)], rows_v.at[0], fsem)
    pcp = pltpu.async_copy(pos_hbm.at[pl.ds(base, SW)], pos_v, psem)

    gcps = []
    icp.wait()
    f0p.wait()
    gcps.append(pltpu.async_copy(tok_hbm.at[idx_v.at[0]],
                                 rows_v.at[0], gsem.at[0], add=True))
    pcp.wait()
    for b in range(1, B):
        def copy_body(j, carry, b=b):
            for d in range(D // LANES):
                sl = pl.ds(d * LANES, LANES)
                rows_v[b, j, sl] = pos_v[j, sl]
            return carry

        lax.fori_loop(0, SW, copy_body, 0)
        gcps.append(pltpu.async_copy(tok_hbm.at[idx_v.at[b]],
                                     rows_v.at[b], gsem.at[b], add=True))

    for b in range(B):
        gcps[b].wait()

    pltpu.async_copy(rows_v, out_hbm.at[:, pl.ds(base, SW), :], ssem).wait()


@jax.jit
def _emb(input_ids, token_table, pos_table):
    mesh = plsc.VectorSubcoreMesh(core_axis_name="c", subcore_axis_name="s")
    return pl.kernel(
        _emb_kernel,
        mesh=mesh,
        out_type=jax.ShapeDtypeStruct((B, S, D), jnp.float32),
        scratch_types=[
            pltpu.VMEM((B, SW), jnp.int32),
            pltpu.VMEM((B, SW, D), jnp.float32),
            pltpu.VMEM((SW, D), jnp.float32),
            pltpu.SemaphoreType.DMA,
            pltpu.SemaphoreType.DMA,
            pltpu.SemaphoreType.DMA,
            pltpu.SemaphoreType.DMA((B,)),
            pltpu.SemaphoreType.DMA,
        ],
    )(input_ids, token_table, pos_table)


def kernel(input_ids, token_table, pos_table):
    return _emb(input_ids, token_table, pos_table)
